# baseline (speedup 1.0000x reference)
import functools

import jax
import jax.numpy as jnp
from jax import lax
from jax.experimental import pallas as pl
from jax.experimental.pallas import tpu as pltpu
from jax.experimental.pallas import tpu_sc as plsc

NUM_ENTITIES = 1000000
NUM_RELATIONS = 500
HIDDEN_DIM = 64
GAMMA_FACT = 12.0
EPSILON = 2.0
PI = 3.1415926235897933
EMB_RANGE = (GAMMA_FACT + EPSILON) / HIDDEN_DIM
B = 16384

REL_PAD = 512

_NC = 2
_NS = 16
_NW = _NC * _NS
_PER_W = B // _NW
_CHUNK = 128
_NCHUNK = _PER_W // _CHUNK


def _trig_body(rel_ref, trig_ref):
    phase = rel_ref[...] * (PI / EMB_RANGE)
    trig = jnp.concatenate([jnp.cos(phase), jnp.sin(phase)], axis=1)
    pad = jnp.zeros((REL_PAD - NUM_RELATIONS - 1, 2 * HIDDEN_DIM), jnp.float32)
    trig_ref[...] = jnp.concatenate([trig, pad], axis=0)


def _make_trig_table(relation_embedding):
    return pl.pallas_call(
        _trig_body,
        out_shape=jax.ShapeDtypeStruct((REL_PAD, 2 * HIDDEN_DIM), jnp.float32),
    )(relation_embedding)


def _rsqrt16(x):
    i = lax.bitcast_convert_type(x, jnp.int32)
    y = lax.bitcast_convert_type(jnp.int32(0x5F3759DF) - (i >> 1), jnp.float32)
    xh = x * jnp.float32(0.5)
    for _ in range(1):
        y = y * (jnp.float32(1.5) - xh * y * y)
    return y


def _lane_permute(v, idx):
    return lax.gather(
        v, idx[:, None],
        dimension_numbers=lax.GatherDimensionNumbers(
            offset_dims=(), collapsed_slice_dims=(0,), start_index_map=(0,)),
        slice_sizes=(1,),
        mode=lax.GatherScatterMode.PROMISE_IN_BOUNDS)


def _sc_body(heads_ref, rels_ref, tails_ref, ent_ref, trig_ref,
             out_ref, idx_h, idx_t, idx_r, hbuf, tbuf, gbuf, scores,
             sem_a, sem_b):
    wid = lax.axis_index("s") * _NC + lax.axis_index("c")
    base = wid * _PER_W
    row0 = wid * _NCHUNK
    pltpu.sync_copy(heads_ref.at[pl.ds(row0, _NCHUNK)], idx_h)
    pltpu.sync_copy(tails_ref.at[pl.ds(row0, _NCHUNK)], idx_t)
    pltpu.sync_copy(rels_ref.at[pl.ds(row0, _NCHUNK)], idx_r)
    lane = lax.iota(jnp.int32, 16)
    sems = (sem_a, sem_b)

    def fire(c):
        slot = c & 1
        return (
            pltpu.async_copy(ent_ref.at[idx_h.at[c]], hbuf.at[slot], sems[slot]),
            pltpu.async_copy(ent_ref.at[idx_t.at[c]], tbuf.at[slot], sems[slot]),
            pltpu.async_copy(trig_ref.at[idx_r.at[c]], gbuf.at[slot], sems[slot]),
        )

    def fire_part(c, p, nparts):
        slot = c & 1
        hs = pl.ds(p * (_CHUNK // nparts), _CHUNK // nparts)
        return (
            pltpu.async_copy(ent_ref.at[idx_h.at[c].at[hs]],
                             hbuf.at[slot].at[hs], sems[slot]),
            pltpu.async_copy(ent_ref.at[idx_t.at[c].at[hs]],
                             tbuf.at[slot].at[hs], sems[slot]),
            pltpu.async_copy(trig_ref.at[idx_r.at[c].at[hs]],
                             gbuf.at[slot].at[hs], sems[slot]),
        )

    _NP0 = 4
    cps = {(0, p): fire_part(0, p, _NP0) for p in range(_NP0)}
    for c in range(_NCHUNK):
        slot = c & 1
        if c + 1 < _NCHUNK:
            cps[(c + 1, 0)] = fire(c + 1)

        hb = hbuf.at[slot]
        tb = tbuf.at[slot]
        gb = gbuf.at[slot]

        def group_body(g, carry, c=c, hb=hb, tb=tb, gb=gb):
            def one_sample(s):
                acc = jnp.zeros((16,), jnp.float32)
                for j in range(HIDDEN_DIM // 16):
                    re_sl = pl.ds(j * 16, 16)
                    im_sl = pl.ds(HIDDEN_DIM + j * 16, 16)
                    reh = hb[s, re_sl]
                    imh = hb[s, im_sl]
                    ret = tb[s, re_sl]
                    imt = tb[s, im_sl]
                    cr = gb[s, re_sl]
                    sr = gb[s, im_sl]
                    re_s = reh * cr - imh * sr - ret
                    im_s = reh * sr + imh * cr - imt
                    x = jnp.maximum(re_s * re_s + im_s * im_s,
                                    jnp.float32(1e-24))
                    acc = acc + x * _rsqrt16(x)
                for m in (1, 2, 4, 8):
                    acc = acc + _lane_permute(acc, lane ^ m)
                return jnp.float32(GAMMA_FACT) - acc * jnp.float32(1.00094598)

            def samp_body(i, svec):
                s = g * 16 + i * 2
                v0 = one_sample(s)
                v1 = one_sample(s + 1)
                svec = jnp.where(lane == i * 2, v0, svec)
                return jnp.where(lane == i * 2 + 1, v1, svec)

            svec = lax.fori_loop(0, 8, samp_body, jnp.zeros((16,), jnp.float32))
            scores[pl.ds(c * _CHUNK + g * 16, 16)] = svec
            return carry

        ng = _CHUNK // 16
        if c == 0:
            for p in range(4):
                for cp in cps.pop((0, p)):
                    cp.wait()
                lax.fori_loop(p * ng // 4, (p + 1) * ng // 4, group_body, 0)
        else:
            for cp in cps.pop((c, 0)):
                cp.wait()
            lax.fori_loop(0, ng, group_body, 0)
    pltpu.sync_copy(scores, out_ref.at[pl.ds(base, _PER_W)])


_sc_score = functools.partial(
    pl.kernel,
    mesh=plsc.VectorSubcoreMesh(core_axis_name="c", subcore_axis_name="s"),
    out_type=jax.ShapeDtypeStruct((B,), jnp.float32),
    scratch_types=[
        pltpu.VMEM((_NCHUNK, _CHUNK), jnp.int32),
        pltpu.VMEM((_NCHUNK, _CHUNK), jnp.int32),
        pltpu.VMEM((_NCHUNK, _CHUNK), jnp.int32),
        pltpu.VMEM((2, _CHUNK, 2 * HIDDEN_DIM), jnp.float32),
        pltpu.VMEM((2, _CHUNK, 2 * HIDDEN_DIM), jnp.float32),
        pltpu.VMEM((2, _CHUNK, 2 * HIDDEN_DIM), jnp.float32),
        pltpu.VMEM((_PER_W,), jnp.float32),
        pltpu.SemaphoreType.DMA,
        pltpu.SemaphoreType.DMA,
    ],
)(_sc_body)


@jax.jit
def _impl(heads, rels, tails, entity_embedding, relation_embedding):
    trig = _make_trig_table(relation_embedding)
    scores = _sc_score(
        heads.reshape(B // _CHUNK, _CHUNK),
        rels.reshape(B // _CHUNK, _CHUNK),
        tails.reshape(B // _CHUNK, _CHUNK),
        entity_embedding,
        trig,
    )
    return scores.reshape(B, 1)


def kernel(heads, rels, tails, entity_embedding, relation_embedding):
    return _impl(heads, rels, tails, entity_embedding, relation_embedding)


# speedup vs baseline: 1.0124x; 1.0124x over previous
import functools

import jax
import jax.numpy as jnp
from jax import lax
from jax.experimental import pallas as pl
from jax.experimental.pallas import tpu as pltpu
from jax.experimental.pallas import tpu_sc as plsc

NUM_ENTITIES = 1000000
NUM_RELATIONS = 500
HIDDEN_DIM = 64
GAMMA_FACT = 12.0
EPSILON = 2.0
PI = 3.1415926235897933
EMB_RANGE = (GAMMA_FACT + EPSILON) / HIDDEN_DIM
B = 16384

REL_PAD = 512

_NC = 2
_NS = 16
_NW = _NC * _NS
_PER_W = B // _NW
_CHUNK = 128
_NCHUNK = _PER_W // _CHUNK


def _trig_body(rel_ref, trig_ref):
    phase = rel_ref[...] * (PI / EMB_RANGE)
    trig = jnp.concatenate([jnp.cos(phase), jnp.sin(phase)], axis=1)
    pad = jnp.zeros((REL_PAD - NUM_RELATIONS - 1, 2 * HIDDEN_DIM), jnp.float32)
    trig_ref[...] = jnp.concatenate([trig, pad], axis=0)


def _make_trig_table(relation_embedding):
    return pl.pallas_call(
        _trig_body,
        out_shape=jax.ShapeDtypeStruct((REL_PAD, 2 * HIDDEN_DIM), jnp.float32),
    )(relation_embedding)


def _rsqrt16(x):
    i = lax.bitcast_convert_type(x, jnp.int32)
    y = lax.bitcast_convert_type(jnp.int32(0x5F3759DF) - (i >> 1), jnp.float32)
    xh = x * jnp.float32(0.5)
    for _ in range(1):
        y = y * (jnp.float32(1.5) - xh * y * y)
    return y


def _lane_permute(v, idx):
    return lax.gather(
        v, idx[:, None],
        dimension_numbers=lax.GatherDimensionNumbers(
            offset_dims=(), collapsed_slice_dims=(0,), start_index_map=(0,)),
        slice_sizes=(1,),
        mode=lax.GatherScatterMode.PROMISE_IN_BOUNDS)


def _sc_body(heads_ref, rels_ref, tails_ref, ent_ref, trig_ref,
             out_ref, idx_h, idx_t, idx_r, hbuf, tbuf, gbuf, scores,
             sem_a, sem_b):
    wid = lax.axis_index("s") * _NC + lax.axis_index("c")
    base = wid * _PER_W
    row0 = wid * _NCHUNK
    pltpu.sync_copy(heads_ref.at[pl.ds(row0, _NCHUNK)], idx_h)
    pltpu.sync_copy(tails_ref.at[pl.ds(row0, _NCHUNK)], idx_t)
    pltpu.sync_copy(rels_ref.at[pl.ds(row0, _NCHUNK)], idx_r)
    lane = lax.iota(jnp.int32, 16)
    sems = (sem_a, sem_b)

    def fire(c):
        slot = c & 1
        return (
            pltpu.async_copy(ent_ref.at[idx_h.at[c]], hbuf.at[slot], sems[slot]),
            pltpu.async_copy(ent_ref.at[idx_t.at[c]], tbuf.at[slot], sems[slot]),
            pltpu.async_copy(trig_ref.at[idx_r.at[c]], gbuf.at[slot], sems[slot]),
        )

    def fire_part(c, p, nparts):
        slot = c & 1
        hs = pl.ds(p * (_CHUNK // nparts), _CHUNK // nparts)
        return (
            pltpu.async_copy(ent_ref.at[idx_h.at[c].at[hs]],
                             hbuf.at[slot].at[hs], sems[slot]),
            pltpu.async_copy(ent_ref.at[idx_t.at[c].at[hs]],
                             tbuf.at[slot].at[hs], sems[slot]),
            pltpu.async_copy(trig_ref.at[idx_r.at[c].at[hs]],
                             gbuf.at[slot].at[hs], sems[slot]),
        )

    _NP0 = 4
    cps = {(0, p): fire_part(0, p, _NP0) for p in range(_NP0)}
    for c in range(_NCHUNK):
        slot = c & 1
        if c + 1 < _NCHUNK:
            cps[(c + 1, 0)] = fire(c + 1)

        hb = hbuf.at[slot]
        tb = tbuf.at[slot]
        gb = gbuf.at[slot]

        def group_body(g, carry, c=c, hb=hb, tb=tb, gb=gb):
            def samp_body(i, svec):
                s = g * 16 + i
                acc = jnp.zeros((16,), jnp.float32)
                for j in range(HIDDEN_DIM // 16):
                    re_sl = pl.ds(j * 16, 16)
                    im_sl = pl.ds(HIDDEN_DIM + j * 16, 16)
                    reh = hb[s, re_sl]
                    imh = hb[s, im_sl]
                    ret = tb[s, re_sl]
                    imt = tb[s, im_sl]
                    cr = gb[s, re_sl]
                    sr = gb[s, im_sl]
                    re_s = reh * cr - imh * sr - ret
                    im_s = reh * sr + imh * cr - imt
                    x = jnp.maximum(re_s * re_s + im_s * im_s,
                                    jnp.float32(1e-24))
                    acc = acc + x * _rsqrt16(x)
                for m in (1, 2, 4, 8):
                    acc = acc + _lane_permute(acc, lane ^ m)
                val = jnp.float32(GAMMA_FACT) - acc * jnp.float32(1.00094598)
                return jnp.where(lane == i, val, svec)

            svec = lax.fori_loop(0, 16, samp_body, jnp.zeros((16,), jnp.float32))
            scores[pl.ds(c * _CHUNK + g * 16, 16)] = svec
            return carry

        ng = _CHUNK // 16
        if c == 0:
            for p in range(4):
                for cp in cps.pop((0, p)):
                    cp.wait()
                lax.fori_loop(p * ng // 4, (p + 1) * ng // 4, group_body, 0)
        else:
            for cp in cps.pop((c, 0)):
                cp.wait()
            lax.fori_loop(0, ng, group_body, 0)
    pltpu.sync_copy(scores, out_ref.at[pl.ds(base, _PER_W)])


_sc_score = functools.partial(
    pl.kernel,
    mesh=plsc.VectorSubcoreMesh(core_axis_name="c", subcore_axis_name="s"),
    out_type=jax.ShapeDtypeStruct((B,), jnp.float32),
    scratch_types=[
        pltpu.VMEM((_NCHUNK, _CHUNK), jnp.int32),
        pltpu.VMEM((_NCHUNK, _CHUNK), jnp.int32),
        pltpu.VMEM((_NCHUNK, _CHUNK), jnp.int32),
        pltpu.VMEM((2, _CHUNK, 2 * HIDDEN_DIM), jnp.float32),
        pltpu.VMEM((2, _CHUNK, 2 * HIDDEN_DIM), jnp.float32),
        pltpu.VMEM((2, _CHUNK, 2 * HIDDEN_DIM), jnp.float32),
        pltpu.VMEM((_PER_W,), jnp.float32),
        pltpu.SemaphoreType.DMA,
        pltpu.SemaphoreType.DMA,
    ],
)(_sc_body)


@jax.jit
def _impl(heads, rels, tails, entity_embedding, relation_embedding):
    trig = _make_trig_table(relation_embedding)
    scores = _sc_score(
        heads.reshape(B // _CHUNK, _CHUNK),
        rels.reshape(B // _CHUNK, _CHUNK),
        tails.reshape(B // _CHUNK, _CHUNK),
        entity_embedding,
        trig,
    )
    return scores.reshape(B, 1)


def kernel(heads, rels, tails, entity_embedding, relation_embedding):
    return _impl(heads, rels, tails, entity_embedding, relation_embedding)
